# baseline (speedup 1.0000x reference)
import jax
from jax import lax
import jax.numpy as jnp
from jax.experimental import pallas as pl
from jax.experimental.pallas import tpu as pltpu
from jax.experimental.pallas import tpu_sc as plsc

NC = 2
NS = 16
NW = NC * NS
W = 128
CHUNK = 160


def kernel(visit_segments, table):
    batch, hist = visit_segments.shape
    vocab, embed = table.shape
    n = batch * hist
    per_w = n // NW
    n_win = per_w // W
    n_chunks = n_win // CHUNK

    idx = visit_segments.reshape(n // W, W).astype(jnp.int32)

    @pl.kernel(
        out_type=jax.ShapeDtypeStruct((n, embed), table.dtype),
        mesh=plsc.VectorSubcoreMesh(core_axis_name="c", subcore_axis_name="s"),
        scratch_types=[
            pltpu.VMEM((CHUNK, W), jnp.int32),
            pltpu.VMEM((W, embed), table.dtype),
            pltpu.VMEM((W, embed), table.dtype),
            pltpu.SemaphoreType.DMA,
            pltpu.SemaphoreType.DMA,
        ],
    )
    def gather_kernel(table_hbm, idx_hbm, out_hbm, idx_v, rows0, rows1, w0, w1):
        wid = lax.axis_index("s") * NC + lax.axis_index("c")
        base_win = wid * n_win

        def out_slice(g):
            return out_hbm.at[pl.ds((base_win + g) * W, W)]

        @pl.loop(0, n_chunks)
        def _(c):
            c0 = c * CHUNK
            pltpu.sync_copy(idx_hbm.at[pl.ds(base_win + c0, CHUNK)], idx_v)

            pltpu.sync_copy(table_hbm.at[idx_v.at[0]], rows0)
            pltpu.async_copy(rows0, out_slice(c0), w0)
            pltpu.sync_copy(table_hbm.at[idx_v.at[1]], rows1)
            pltpu.async_copy(rows1, out_slice(c0 + 1), w1)

            @pl.loop(2, CHUNK, step=2)
            def _(v):
                g = c0 + v
                pltpu.make_async_copy(rows0, out_slice(g), w0).wait()
                pltpu.sync_copy(table_hbm.at[idx_v.at[v]], rows0)
                pltpu.async_copy(rows0, out_slice(g), w0)
                pltpu.make_async_copy(rows1, out_slice(g + 1), w1).wait()
                pltpu.sync_copy(table_hbm.at[idx_v.at[v + 1]], rows1)
                pltpu.async_copy(rows1, out_slice(g + 1), w1)

            pltpu.make_async_copy(rows0, out_slice(c0), w0).wait()
            pltpu.make_async_copy(rows1, out_slice(c0 + 1), w1).wait()

    out = gather_kernel(table, idx)
    return out.reshape(batch, hist, embed)


# speedup vs baseline: 1.0219x; 1.0219x over previous
import jax
from jax import lax
import jax.numpy as jnp
from jax.experimental import pallas as pl
from jax.experimental.pallas import tpu as pltpu
from jax.experimental.pallas import tpu_sc as plsc

NC = 2
NS = 16
NW = NC * NS
W = 128
NBUF = 4
CHUNK = 160


def kernel(visit_segments, table):
    batch, hist = visit_segments.shape
    vocab, embed = table.shape
    n = batch * hist
    per_w = n // NW
    n_win = per_w // W
    n_chunks = n_win // CHUNK

    idx = visit_segments.reshape(n // W, W).astype(jnp.int32)

    scratch = [pltpu.VMEM((CHUNK, W), jnp.int32)]
    scratch += [pltpu.VMEM((W, embed), table.dtype) for _ in range(NBUF)]
    scratch += [pltpu.SemaphoreType.DMA for _ in range(2 * NBUF)]

    @pl.kernel(
        out_type=jax.ShapeDtypeStruct((n, embed), table.dtype),
        mesh=plsc.VectorSubcoreMesh(core_axis_name="c", subcore_axis_name="s"),
        scratch_types=scratch,
    )
    def gather_kernel(table_hbm, idx_hbm, out_hbm, idx_v, *bufs_and_sems):
        rows = bufs_and_sems[:NBUF]
        gsem = bufs_and_sems[NBUF:2 * NBUF]
        wsem = bufs_and_sems[2 * NBUF:]
        wid = lax.axis_index("s") * NC + lax.axis_index("c")
        base_win = wid * n_win

        def out_slice(g):
            return out_hbm.at[pl.ds((base_win + g) * W, W)]

        def start_gather(j, v):
            pltpu.async_copy(table_hbm.at[idx_v.at[v]], rows[j], gsem[j])

        def wait_gather(j):
            pltpu.make_async_copy(table_hbm.at[idx_v.at[0]], rows[j],
                                  gsem[j]).wait()

        def start_write(j, g):
            pltpu.async_copy(rows[j], out_slice(g), wsem[j])

        def wait_write(j, g):
            pltpu.make_async_copy(rows[j], out_slice(g), wsem[j]).wait()

        @pl.loop(0, n_chunks)
        def _(c):
            c0 = c * CHUNK
            pltpu.sync_copy(idx_hbm.at[pl.ds(base_win + c0, CHUNK)], idx_v)

            for j in range(NBUF):
                start_gather(j, j)

            @pl.loop(NBUF, CHUNK, step=NBUF)
            def _(v):
                for j in range(NBUF):
                    wait_gather(j)
                    start_write(j, c0 + v - NBUF + j)
                for j in range(NBUF):
                    wait_write(j, c0 + v - NBUF + j)
                    start_gather(j, v + j)

            for j in range(NBUF):
                wait_gather(j)
                start_write(j, c0 + CHUNK - NBUF + j)
            for j in range(NBUF):
                wait_write(j, c0 + CHUNK - NBUF + j)

    out = gather_kernel(table, idx)
    return out.reshape(batch, hist, embed)
